# baseline (speedup 1.0000x reference)
import functools

import jax
import jax.numpy as jnp
from jax import lax
from jax.experimental import pallas as pl
from jax.experimental.pallas import tpu as pltpu
from jax.experimental.pallas import tpu_sc as plsc

B = 16384
L = 20
D = 32
NC, NS = 2, 16
NW = NC * NS
CHUNK = B // NW
SUB = 128
NSUB = CHUNK // SUB


def _sc_body(ttl_hbm, tok_hbm, ttable_hbm, xtable_hbm, out_hbm,
             tok_idx, tok_flat, ttl_idx, rows, trows, packed,
             inv_buf, npad_buf, row0, sem):
    wid = lax.axis_index("s") * NC + lax.axis_index("c")
    base = wid * CHUNK

    pltpu.sync_copy(ttl_hbm.at[pl.ds(base, CHUNK)], ttl_idx)
    pltpu.sync_copy(tok_hbm.at[pl.ds(base, CHUNK), :], tok_idx)
    pltpu.sync_copy(xtable_hbm.at[0], row0)
    r0a = row0[pl.ds(0, 16)]
    r0b = row0[pl.ds(16, 16)]
    lane = lax.iota(jnp.int32, 16)
    zero16 = jnp.zeros((16,), jnp.float32)

    def cbody(g, carry):
        brow = g * 16 + lane
        flat0 = brow * L
        cnt = zero16
        for l in range(L):
            ids = plsc.load_gather(
                tok_idx, [brow, jnp.full((16,), l, jnp.int32)])
            plsc.store_scatter(tok_flat, [flat0 + l], ids)
            cnt = cnt + (ids != 0).astype(jnp.float32)
        inv_buf[pl.ds(g * 16, 16)] = (
            jnp.ones((16,), jnp.float32) / jnp.maximum(cnt, 1.0))
        npad_buf[pl.ds(g * 16, 16)] = (
            jnp.full((16,), float(L), jnp.float32) - cnt)
        return carry

    lax.fori_loop(0, CHUNK // 16, cbody, 0)

    for s in range(NSUB):
        sbase = s * SUB
        cps = [pltpu.async_copy(
            ttable_hbm.at[ttl_idx.at[pl.ds(sbase, SUB)]], trows, sem)]
        for g in range(SUB * L // 128):
            cps.append(pltpu.async_copy(
                xtable_hbm.at[tok_flat.at[pl.ds(sbase * L + g * 128, 128)]],
                rows.at[pl.ds(g * 128, 128)], sem))
        for c in cps:
            c.wait()

        def body(i, carry):
            i16 = jnp.broadcast_to(sbase + i, (16,))
            inv = plsc.load_gather(inv_buf, [i16])
            npad = plsc.load_gather(npad_buf, [i16])
            s0, s1 = zero16, zero16
            for l in range(L):
                s0 = s0 + rows[i * L + l, pl.ds(0, 16)]
                s1 = s1 + rows[i * L + l, pl.ds(16, 16)]
            packed[i, pl.ds(0, 16)] = trows[i, pl.ds(0, 16)]
            packed[i, pl.ds(16, 16)] = trows[i, pl.ds(16, 16)]
            packed[i, pl.ds(32, 16)] = (s0 - npad * r0a) * inv
            packed[i, pl.ds(48, 16)] = (s1 - npad * r0b) * inv
            return carry

        lax.fori_loop(0, SUB, body, 0)
        pltpu.sync_copy(packed, out_hbm.at[pl.ds(base + sbase, SUB), :])


@jax.jit
def _product_model(title_ids, token_ids, title_table, text_table):
    mesh = plsc.VectorSubcoreMesh(core_axis_name="c", subcore_axis_name="s")
    f = functools.partial(
        pl.kernel,
        out_type=jax.ShapeDtypeStruct((B, 2 * D), jnp.float32),
        mesh=mesh,
        scratch_types=[
            pltpu.VMEM((CHUNK, D), jnp.int32),
            pltpu.VMEM((CHUNK * L,), jnp.int32),
            pltpu.VMEM((CHUNK,), jnp.int32),
            pltpu.VMEM((SUB * L, D), jnp.float32),
            pltpu.VMEM((SUB, D), jnp.float32),
            pltpu.VMEM((SUB, 2 * D), jnp.float32),
            pltpu.VMEM((CHUNK,), jnp.float32),
            pltpu.VMEM((CHUNK,), jnp.float32),
            pltpu.VMEM((D,), jnp.float32),
            pltpu.SemaphoreType.DMA,
        ],
        compiler_params=pltpu.CompilerParams(use_tc_tiling_on_sc=False,
                                             needs_layout_passes=False),
    )(_sc_body)
    return f(title_ids, token_ids, title_table, text_table)


def kernel(title_ids, token_ids, title_table, text_table):
    tok = jnp.pad(token_ids.astype(jnp.int32), ((0, 0), (0, D - L)))
    return _product_model(title_ids.astype(jnp.int32), tok,
                          title_table, text_table)


# speedup vs baseline: 1.1613x; 1.1613x over previous
import functools

import jax
import jax.numpy as jnp
from jax import lax
from jax.experimental import pallas as pl
from jax.experimental.pallas import tpu as pltpu
from jax.experimental.pallas import tpu_sc as plsc

B = 16384
L = 20
D = 32
NC, NS = 2, 16
NW = NC * NS
CHUNK = B // NW
SUB = 128
NSUB = CHUNK // SUB

_MESH = dict(core_axis_name="c", subcore_axis_name="s")
_PARAMS = pltpu.CompilerParams(use_tc_tiling_on_sc=False,
                               needs_layout_passes=False)


def _wid():
    return lax.axis_index("s") * NC + lax.axis_index("c")


def _pool_body(tok_hbm, xtable_hbm, out_hbm,
               tok_idx, tok_flat, rows, packed, inv_buf, npad_buf, row0, sem):
    base = _wid() * CHUNK

    pltpu.sync_copy(tok_hbm.at[pl.ds(base, CHUNK), :], tok_idx)
    pltpu.sync_copy(xtable_hbm.at[0], row0)
    r0a = row0[pl.ds(0, 16)]
    r0b = row0[pl.ds(16, 16)]
    lane = lax.iota(jnp.int32, 16)
    zero16 = jnp.zeros((16,), jnp.float32)

    def cbody(g, carry):
        brow = g * 16 + lane
        flat0 = brow * L
        cnt = zero16
        for l in range(L):
            ids = plsc.load_gather(
                tok_idx, [brow, jnp.full((16,), l, jnp.int32)])
            plsc.store_scatter(tok_flat, [flat0 + l], ids)
            cnt = cnt + (ids != 0).astype(jnp.float32)
        inv_buf[pl.ds(g * 16, 16)] = (
            jnp.ones((16,), jnp.float32) / jnp.maximum(cnt, 1.0))
        npad_buf[pl.ds(g * 16, 16)] = (
            jnp.full((16,), float(L), jnp.float32) - cnt)
        return carry

    lax.fori_loop(0, CHUNK // 16, cbody, 0)

    for s in range(NSUB):
        sbase = s * SUB
        cps = []
        for g in range(SUB * L // 128):
            cps.append(pltpu.async_copy(
                xtable_hbm.at[tok_flat.at[pl.ds(sbase * L + g * 128, 128)]],
                rows.at[pl.ds(g * 128, 128)], sem))
        for c in cps:
            c.wait()

        def body(i, carry):
            i16 = jnp.broadcast_to(sbase + i, (16,))
            inv = plsc.load_gather(inv_buf, [i16])
            npad = plsc.load_gather(npad_buf, [i16])
            s0, s1 = zero16, zero16
            for l in range(L):
                s0 = s0 + rows[i * L + l, pl.ds(0, 16)]
                s1 = s1 + rows[i * L + l, pl.ds(16, 16)]
            packed[i, pl.ds(0, 16)] = (s0 - npad * r0a) * inv
            packed[i, pl.ds(16, 16)] = (s1 - npad * r0b) * inv
            return carry

        lax.fori_loop(0, SUB, body, 0)
        pltpu.sync_copy(packed, out_hbm.at[pl.ds(base + sbase, SUB), :])


def _title_body(ttl_hbm, ttable_hbm, out_hbm, ttl_idx, trows, sem):
    base = _wid() * CHUNK
    pltpu.sync_copy(ttl_hbm.at[pl.ds(base, CHUNK)], ttl_idx)
    cps = [pltpu.async_copy(
        ttable_hbm.at[ttl_idx.at[pl.ds(g * 128, 128)]],
        trows.at[pl.ds(g * 128, 128)], sem) for g in range(CHUNK // 128)]
    for c in cps:
        c.wait()
    pltpu.sync_copy(trows, out_hbm.at[pl.ds(base, CHUNK), :])


@jax.jit
def _product_model(title_ids, token_ids, title_table, text_table):
    pooled = functools.partial(
        pl.kernel,
        out_type=jax.ShapeDtypeStruct((B, D), jnp.float32),
        mesh=plsc.VectorSubcoreMesh(**_MESH),
        scratch_types=[
            pltpu.VMEM((CHUNK, L), jnp.int32),
            pltpu.VMEM((CHUNK * L,), jnp.int32),
            pltpu.VMEM((SUB * L, D), jnp.float32),
            pltpu.VMEM((SUB, D), jnp.float32),
            pltpu.VMEM((CHUNK,), jnp.float32),
            pltpu.VMEM((CHUNK,), jnp.float32),
            pltpu.VMEM((D,), jnp.float32),
            pltpu.SemaphoreType.DMA,
        ],
        compiler_params=_PARAMS,
    )(_pool_body)(token_ids, text_table)

    title = functools.partial(
        pl.kernel,
        out_type=jax.ShapeDtypeStruct((B, D), jnp.float32),
        mesh=plsc.VectorSubcoreMesh(**_MESH),
        scratch_types=[
            pltpu.VMEM((CHUNK,), jnp.int32),
            pltpu.VMEM((CHUNK, D), jnp.float32),
            pltpu.SemaphoreType.DMA,
        ],
        compiler_params=_PARAMS,
    )(_title_body)(title_ids, title_table)

    return jnp.concatenate([title, pooled], axis=1)


def kernel(title_ids, token_ids, title_table, text_table):
    return _product_model(title_ids.astype(jnp.int32),
                          token_ids.astype(jnp.int32),
                          title_table, text_table)


# speedup vs baseline: 1.2060x; 1.0385x over previous
import functools

import jax
import jax.numpy as jnp
from jax import lax
from jax.experimental import pallas as pl
from jax.experimental.pallas import tpu as pltpu
from jax.experimental.pallas import tpu_sc as plsc

B = 16384
L = 20
D = 32
NC, NS = 2, 16
NW = NC * NS
CHUNK = B // NW
SUB = 64
NSUB = CHUNK // SUB
NGATH = SUB * L // 128

_MESH = dict(core_axis_name="c", subcore_axis_name="s")
_PARAMS = pltpu.CompilerParams(use_tc_tiling_on_sc=False,
                               needs_layout_passes=False)


def _wid():
    return lax.axis_index("s") * NC + lax.axis_index("c")


def _pool_body(tok_hbm, xtable_hbm, out_hbm,
               tok_idx, tok_flat, rows0, rows1, packed,
               inv_buf, npad_buf, row0, sem0, sem1, osem):
    base = _wid() * CHUNK

    pltpu.sync_copy(tok_hbm.at[pl.ds(base, CHUNK), :], tok_idx)
    pltpu.sync_copy(xtable_hbm.at[0], row0)
    r0a = row0[pl.ds(0, 16)]
    r0b = row0[pl.ds(16, 16)]
    lane = lax.iota(jnp.int32, 16)
    zero16 = jnp.zeros((16,), jnp.float32)

    def cbody(g, carry):
        brow = g * 16 + lane
        flat0 = brow * L
        cnt = zero16
        for l in range(L):
            ids = plsc.load_gather(
                tok_idx, [brow, jnp.full((16,), l, jnp.int32)])
            plsc.store_scatter(tok_flat, [flat0 + l], ids)
            cnt = cnt + (ids != 0).astype(jnp.float32)
        inv_buf[pl.ds(g * 16, 16)] = (
            jnp.ones((16,), jnp.float32) / jnp.maximum(cnt, 1.0))
        npad_buf[pl.ds(g * 16, 16)] = (
            jnp.full((16,), float(L), jnp.float32) - cnt)
        return carry

    lax.fori_loop(0, CHUNK // 16, cbody, 0)

    rows = (rows0, rows1)
    sems = (sem0, sem1)

    def fire(s, k):
        return [pltpu.async_copy(
            xtable_hbm.at[tok_flat.at[pl.ds(s * SUB * L + g * 128, 128)]],
            rows[k].at[pl.ds(g * 128, 128)], sems[k])
            for g in range(NGATH)]

    inflight = {0: fire(0, 0)}
    out_cp = None
    for s in range(NSUB):
        k = s % 2
        if s + 1 < NSUB:
            inflight[s + 1] = fire(s + 1, (s + 1) % 2)
        for c in inflight.pop(s):
            c.wait()
        if out_cp is not None:
            out_cp.wait()

        sbase = s * SUB

        def body(i, carry):
            i16 = jnp.broadcast_to(sbase + i, (16,))
            inv = plsc.load_gather(inv_buf, [i16])
            npad = plsc.load_gather(npad_buf, [i16])
            s0, s1 = zero16, zero16
            for l in range(L):
                s0 = s0 + rows[k][i * L + l, pl.ds(0, 16)]
                s1 = s1 + rows[k][i * L + l, pl.ds(16, 16)]
            packed[i, pl.ds(0, 16)] = (s0 - npad * r0a) * inv
            packed[i, pl.ds(16, 16)] = (s1 - npad * r0b) * inv
            return carry

        lax.fori_loop(0, SUB, body, 0)
        out_cp = pltpu.async_copy(
            packed, out_hbm.at[pl.ds(base + sbase, SUB), :], osem)
    out_cp.wait()


def _title_body(ttl_hbm, ttable_hbm, pooled_hbm, out_hbm,
                ttl_idx, trows, prows, packed, sem):
    base = _wid() * CHUNK
    pltpu.sync_copy(ttl_hbm.at[pl.ds(base, CHUNK)], ttl_idx)
    cps = [pltpu.async_copy(
        ttable_hbm.at[ttl_idx.at[pl.ds(g * 128, 128)]],
        trows.at[pl.ds(g * 128, 128)], sem) for g in range(CHUNK // 128)]
    pltpu.sync_copy(pooled_hbm.at[pl.ds(base, CHUNK), :], prows)
    for c in cps:
        c.wait()

    def body(i, carry):
        packed[i, pl.ds(0, 16)] = trows[i, pl.ds(0, 16)]
        packed[i, pl.ds(16, 16)] = trows[i, pl.ds(16, 16)]
        packed[i, pl.ds(32, 16)] = prows[i, pl.ds(0, 16)]
        packed[i, pl.ds(48, 16)] = prows[i, pl.ds(16, 16)]
        return carry

    lax.fori_loop(0, CHUNK, body, 0)
    pltpu.sync_copy(packed, out_hbm.at[pl.ds(base, CHUNK), :])


@jax.jit
def _product_model(title_ids, token_ids, title_table, text_table):
    pooled = functools.partial(
        pl.kernel,
        out_type=jax.ShapeDtypeStruct((B, D), jnp.float32),
        mesh=plsc.VectorSubcoreMesh(**_MESH),
        scratch_types=[
            pltpu.VMEM((CHUNK, L), jnp.int32),
            pltpu.VMEM((CHUNK * L,), jnp.int32),
            pltpu.VMEM((SUB * L, D), jnp.float32),
            pltpu.VMEM((SUB * L, D), jnp.float32),
            pltpu.VMEM((SUB, D), jnp.float32),
            pltpu.VMEM((CHUNK,), jnp.float32),
            pltpu.VMEM((CHUNK,), jnp.float32),
            pltpu.VMEM((D,), jnp.float32),
            pltpu.SemaphoreType.DMA,
            pltpu.SemaphoreType.DMA,
            pltpu.SemaphoreType.DMA,
        ],
        compiler_params=_PARAMS,
    )(_pool_body)(token_ids, text_table)

    out = functools.partial(
        pl.kernel,
        out_type=jax.ShapeDtypeStruct((B, 2 * D), jnp.float32),
        mesh=plsc.VectorSubcoreMesh(**_MESH),
        scratch_types=[
            pltpu.VMEM((CHUNK,), jnp.int32),
            pltpu.VMEM((CHUNK, D), jnp.float32),
            pltpu.VMEM((CHUNK, D), jnp.float32),
            pltpu.VMEM((CHUNK, 2 * D), jnp.float32),
            pltpu.SemaphoreType.DMA,
        ],
        compiler_params=_PARAMS,
    )(_title_body)(title_ids, title_table, pooled)

    return out


def kernel(title_ids, token_ids, title_table, text_table):
    return _product_model(title_ids.astype(jnp.int32),
                          token_ids.astype(jnp.int32),
                          title_table, text_table)
